# baseline (speedup 1.0000x reference)
import functools

import jax
import jax.numpy as jnp
from jax.experimental import pallas as pl
from jax.experimental.pallas import tpu as pltpu


def _matmul_body(x_ref, w_ref, b_ref, z_ref, *, bd):
    j = pl.program_id(0)
    z = jax.lax.dot_general(
        x_ref[...], w_ref[...], (((1,), (1,)), ((), ())),
        preferred_element_type=jnp.float32)
    z_ref[...] = z + b_ref[:, pl.ds(j * bd, bd)]


def _select_body(z_ref, out_ref, *, topk):
    imin = jnp.int32(-(2**31))
    ib = jax.lax.bitcast_convert_type(z_ref[...], jnp.int32)
    skey = jnp.where(ib >= 0, ib, imin - ib - jnp.int32(1))
    out_ref[...] = jax.lax.bitcast_convert_type(skey, jnp.float32)

    def body(it, t):
        shift = jnp.int32(31) - it
        step = jnp.where(shift == 31, imin,
                         jax.lax.shift_left(jnp.int32(1), shift))
        cand = t + step
        s = jax.lax.bitcast_convert_type(out_ref[...], jnp.int32)
        cnt = jnp.sum((s >= cand).astype(jnp.int32), axis=1, keepdims=True)
        return jnp.where(cnt >= topk, cand, t)

    t0 = jnp.full((out_ref.shape[0], 1), imin, jnp.int32)
    t = jax.lax.fori_loop(0, 32, body, t0)

    s = jax.lax.bitcast_convert_type(out_ref[...], jnp.int32)
    mask = s >= t
    zbits = jnp.where(s >= 0, s, imin - s - jnp.int32(1))
    zrec = jax.lax.bitcast_convert_type(zbits, jnp.float32)
    out_ref[...] = jnp.where(mask, jnp.maximum(zrec, 0.0), 0.0)


def _decode_body(s_ref, w_ref, b_ref, out_ref, *, nk):
    k = pl.program_id(1)
    acc = jax.lax.dot_general(
        s_ref[...], w_ref[...], (((1,), (1,)), ((), ())),
        preferred_element_type=jnp.float32)

    @pl.when(k == 0)
    def _():
        out_ref[...] = acc + b_ref[...]

    @pl.when(k != 0)
    def _():
        out_ref[...] += acc


def _topksae_fwd(x, W_enc, b_enc, W_dec, b_dec, *, topk, tmz, bd, tms, tm2,
                 kd, interpret=False):
    n_tok, d_model = x.shape
    d_dict = W_enc.shape[0]
    niz, nj = n_tok // tmz, d_dict // bd
    b_enc2 = b_enc.reshape(1, d_dict)
    z = pl.pallas_call(
        functools.partial(_matmul_body, bd=bd),
        grid=(nj, niz),
        in_specs=[
            pl.BlockSpec((tmz, d_model), lambda j, i: (i, 0)),
            pl.BlockSpec((bd, d_model), lambda j, i: (j, 0)),
            pl.BlockSpec((1, d_dict), lambda j, i: (0, 0)),
        ],
        out_specs=pl.BlockSpec((tmz, bd), lambda j, i: (i, j)),
        out_shape=jax.ShapeDtypeStruct((n_tok, d_dict), jnp.float32),
        compiler_params=pltpu.CompilerParams(
            dimension_semantics=("parallel", "parallel")),
        interpret=interpret,
    )(x, W_enc, b_enc2)

    nis = n_tok // tms
    sparse = pl.pallas_call(
        functools.partial(_select_body, topk=topk),
        grid=(nis,),
        in_specs=[pl.BlockSpec((tms, d_dict), lambda i: (i, 0))],
        out_specs=pl.BlockSpec((tms, d_dict), lambda i: (i, 0)),
        out_shape=jax.ShapeDtypeStruct((n_tok, d_dict), jnp.float32),
        compiler_params=pltpu.CompilerParams(
            dimension_semantics=("parallel",)),
        interpret=interpret,
    )(z)

    ni2, nk = n_tok // tm2, d_dict // kd
    b_dec2 = b_dec.reshape(1, d_model)
    x_hat = pl.pallas_call(
        functools.partial(_decode_body, nk=nk),
        grid=(ni2, nk),
        in_specs=[
            pl.BlockSpec((tm2, kd), lambda i, k: (i, k)),
            pl.BlockSpec((d_model, kd), lambda i, k: (0, k)),
            pl.BlockSpec((1, d_model), lambda i, k: (0, 0)),
        ],
        out_specs=pl.BlockSpec((tm2, d_model), lambda i, k: (i, 0)),
        out_shape=jax.ShapeDtypeStruct((n_tok, d_model), jnp.float32),
        compiler_params=pltpu.CompilerParams(
            dimension_semantics=("parallel", "arbitrary")),
        interpret=interpret,
    )(sparse, W_dec, b_dec2)
    return (x_hat, sparse)


def kernel(x, W_enc, b_enc, W_dec, b_dec):
    return _topksae_fwd(x, W_enc, b_enc, W_dec, b_dec,
                        topk=64, tmz=512, bd=1024, tms=128, tm2=512, kd=1024)


# speedup vs baseline: 1.0846x; 1.0846x over previous
import functools

import jax
import jax.numpy as jnp
from jax.experimental import pallas as pl
from jax.experimental.pallas import tpu as pltpu


def _matmul_body(x_ref, w_ref, b_ref, z_ref, *, bd):
    j = pl.program_id(0)
    z = jax.lax.dot_general(
        x_ref[...], w_ref[...], (((1,), (1,)), ((), ())),
        preferred_element_type=jnp.float32)
    z_ref[...] = z + b_ref[:, pl.ds(j * bd, bd)]


def _select_body(z_ref, out_ref, *, topk):
    imin = jnp.int32(-(2**31))
    ib = jax.lax.bitcast_convert_type(z_ref[...], jnp.int32)
    skey = jnp.where(ib >= 0, ib, imin - ib - jnp.int32(1))
    out_ref[...] = jax.lax.bitcast_convert_type(skey, jnp.float32)

    def body(it, t):
        shift = jnp.int32(31) - it
        step = jnp.where(shift == 31, imin,
                         jax.lax.shift_left(jnp.int32(1), shift))
        cand = t + step
        s = jax.lax.bitcast_convert_type(out_ref[...], jnp.int32)
        cnt = jnp.sum((s >= cand).astype(jnp.int32), axis=1, keepdims=True)
        return jnp.where(cnt >= topk, cand, t)

    t0 = jnp.full((out_ref.shape[0], 1), imin, jnp.int32)
    t = jax.lax.fori_loop(0, 32, body, t0)

    s = jax.lax.bitcast_convert_type(out_ref[...], jnp.int32)
    mask = s >= t
    zbits = jnp.where(s >= 0, s, imin - s - jnp.int32(1))
    zrec = jax.lax.bitcast_convert_type(zbits, jnp.float32)
    out_ref[...] = jnp.where(mask, jnp.maximum(zrec, 0.0), 0.0)


def _decode_body(s_ref, w_ref, b_ref, out_ref, *, nk):
    k = pl.program_id(1)
    acc = jax.lax.dot_general(
        s_ref[...], w_ref[...], (((1,), (1,)), ((), ())),
        preferred_element_type=jnp.float32)

    @pl.when(k == 0)
    def _():
        out_ref[...] = acc + b_ref[...]

    @pl.when(k != 0)
    def _():
        out_ref[...] += acc


def _topksae_fwd(x, W_enc, b_enc, W_dec, b_dec, *, topk, tmz, bd, tms, tm2,
                 kd, interpret=False):
    n_tok, d_model = x.shape
    d_dict = W_enc.shape[0]
    niz, nj = n_tok // tmz, d_dict // bd
    b_enc2 = b_enc.reshape(1, d_dict)
    z = pl.pallas_call(
        functools.partial(_matmul_body, bd=bd),
        grid=(nj, niz),
        in_specs=[
            pl.BlockSpec((tmz, d_model), lambda j, i: (i, 0)),
            pl.BlockSpec((bd, d_model), lambda j, i: (j, 0)),
            pl.BlockSpec((1, d_dict), lambda j, i: (0, 0)),
        ],
        out_specs=pl.BlockSpec((tmz, bd), lambda j, i: (i, j)),
        out_shape=jax.ShapeDtypeStruct((n_tok, d_dict), jnp.float32),
        compiler_params=pltpu.CompilerParams(
            dimension_semantics=("parallel", "parallel")),
        interpret=interpret,
    )(x, W_enc, b_enc2)

    nis = n_tok // tms
    sparse = pl.pallas_call(
        functools.partial(_select_body, topk=topk),
        grid=(nis,),
        in_specs=[pl.BlockSpec((tms, d_dict), lambda i: (i, 0))],
        out_specs=pl.BlockSpec((tms, d_dict), lambda i: (i, 0)),
        out_shape=jax.ShapeDtypeStruct((n_tok, d_dict), jnp.float32),
        compiler_params=pltpu.CompilerParams(
            dimension_semantics=("parallel",)),
        interpret=interpret,
    )(z)

    ni2, nk = n_tok // tm2, d_dict // kd
    b_dec2 = b_dec.reshape(1, d_model)
    x_hat = pl.pallas_call(
        functools.partial(_decode_body, nk=nk),
        grid=(ni2, nk),
        in_specs=[
            pl.BlockSpec((tm2, kd), lambda i, k: (i, k)),
            pl.BlockSpec((d_model, kd), lambda i, k: (0, k)),
            pl.BlockSpec((1, d_model), lambda i, k: (0, 0)),
        ],
        out_specs=pl.BlockSpec((tm2, d_model), lambda i, k: (i, 0)),
        out_shape=jax.ShapeDtypeStruct((n_tok, d_model), jnp.float32),
        compiler_params=pltpu.CompilerParams(
            dimension_semantics=("parallel", "arbitrary")),
        interpret=interpret,
    )(sparse, W_dec, b_dec2)
    return (x_hat, sparse)


def kernel(x, W_enc, b_enc, W_dec, b_dec):
    return _topksae_fwd(x, W_enc, b_enc, W_dec, b_dec,
                        topk=64, tmz=512, bd=2048, tms=128, tm2=1024, kd=1024)
